# Initial kernel scaffold; baseline (speedup 1.0000x reference)
import jax
import jax.numpy as jnp
from jax.experimental import pallas as pl


def kernel(x, edge_index, weights, biases):
    raise NotImplementedError("write your pallas kernel here")


# speedup vs baseline: 8.7240x; 8.7240x over previous
import functools

import jax
import jax.numpy as jnp
from jax import lax
from jax.experimental import pallas as pl
from jax.experimental.pallas import tpu as pltpu
from jax.experimental.pallas import tpu_sc as plsc

N = 10000
E = 320000
D = 128
NC = 2
NS = 16
NW = NC * NS
EW = E // NW
CH = 80
ITERS = EW // CH
ROWS_PER_SUB = N // NS
STG = 125
NSTG = ROWS_PER_SUB // STG
DEG_CH = 624
DEG_REM = N - NS * DEG_CH

_mesh = plsc.VectorSubcoreMesh(core_axis_name="c", subcore_axis_name="s")


def _fill1d(ref, n, value):
    vec = jnp.full((16,), value, jnp.float32)

    def body(i, carry):
        ref[pl.ds(i * 16, 16)] = vec
        return carry

    lax.fori_loop(0, n // 16, body, 0)


def _fill2d(ref, rows, value):
    vec = jnp.full((16,), value, jnp.float32)

    def rbody(r, carry):
        def cbody(c, carry2):
            ref[r, pl.ds(c * 16, 16)] = vec
            return carry2

        return lax.fori_loop(0, D // 16, cbody, carry)

    lax.fori_loop(0, rows, rbody, 0)


@functools.partial(
    pl.kernel,
    out_type=jax.ShapeDtypeStruct((NC * N,), jnp.float32),
    mesh=_mesh,
    compiler_params=pltpu.CompilerParams(use_tc_tiling_on_sc=False),
    scratch_types=[
        pltpu.VMEM_SHARED((N,), jnp.float32),
        pltpu.VMEM((CH,), jnp.int32),
        pltpu.VMEM((CH,), jnp.float32),
        pltpu.VMEM((DEG_CH,), jnp.float32),
    ],
)
def _deg_kernel(col_hbm, deg_out, acc, colbuf, onesbuf, stage):
    c = lax.axis_index("c")
    s = lax.axis_index("s")
    wid = c * NS + s

    _fill1d(stage, DEG_CH, 0.0)
    _fill1d(onesbuf, CH, 1.0)
    pltpu.sync_copy(stage, acc.at[pl.ds(s * DEG_CH, DEG_CH)])

    @pl.when(s == NS - 1)
    def _zero_tail():
        pltpu.sync_copy(stage.at[pl.ds(0, DEG_REM)],
                        acc.at[pl.ds(NS * DEG_CH, DEG_REM)])

    plsc.subcore_barrier()

    def body(i, carry):
        base = wid * EW + i * CH
        pltpu.sync_copy(col_hbm.at[pl.ds(base, CH)], colbuf)
        pltpu.sync_copy(onesbuf, acc.at[colbuf], add=True)
        return carry

    lax.fori_loop(0, ITERS, body, 0)
    plsc.subcore_barrier()

    pltpu.sync_copy(acc.at[pl.ds(s * DEG_CH, DEG_CH)], stage)
    pltpu.sync_copy(stage, deg_out.at[pl.ds(c * N + s * DEG_CH, DEG_CH)])

    @pl.when(s == NS - 1)
    def _copy_tail():
        pltpu.sync_copy(acc.at[pl.ds(NS * DEG_CH, DEG_REM)],
                        stage.at[pl.ds(0, DEG_REM)])
        pltpu.sync_copy(stage.at[pl.ds(0, DEG_REM)],
                        deg_out.at[pl.ds(c * N + NS * DEG_CH, DEG_REM)])


@functools.partial(
    pl.kernel,
    out_type=jax.ShapeDtypeStruct((NC * N, D), jnp.float32),
    mesh=_mesh,
    compiler_params=pltpu.CompilerParams(use_tc_tiling_on_sc=False),
    scratch_types=[
        pltpu.VMEM_SHARED((N, D), jnp.float32),
        pltpu.VMEM((CH,), jnp.int32),
        pltpu.VMEM((CH,), jnp.int32),
        pltpu.VMEM((CH, D), jnp.float32),
        pltpu.VMEM((STG, D), jnp.float32),
        pltpu.SemaphoreType.DMA,
    ],
)
def _agg_kernel(p_hbm, row_hbm, col_hbm, g_out,
                acc, rowbuf, colbuf, buf, stage, sem):
    c = lax.axis_index("c")
    s = lax.axis_index("s")
    wid = c * NS + s

    _fill2d(stage, STG, 0.0)

    def zbody(k, carry):
        pltpu.sync_copy(stage, acc.at[pl.ds(s * ROWS_PER_SUB + k * STG, STG)])
        return carry

    lax.fori_loop(0, NSTG, zbody, 0)
    plsc.subcore_barrier()

    def body(i, carry):
        base = wid * EW + i * CH
        pltpu.sync_copy(row_hbm.at[pl.ds(base, CH)], rowbuf)
        pltpu.sync_copy(col_hbm.at[pl.ds(base, CH)], colbuf)
        pltpu.async_copy(p_hbm.at[rowbuf], buf, sem).wait()
        pltpu.sync_copy(buf, acc.at[colbuf], add=True)
        return carry

    lax.fori_loop(0, ITERS, body, 0)
    plsc.subcore_barrier()

    def obody(k, carry):
        base = s * ROWS_PER_SUB + k * STG
        pltpu.sync_copy(acc.at[pl.ds(base, STG)], stage)
        pltpu.sync_copy(stage, g_out.at[pl.ds(c * N + base, STG)])
        return carry

    lax.fori_loop(0, NSTG, obody, 0)


BN = 1000


def _tc_body(variant, *refs):
    if variant == "res":
        dis_ref, g_ref, p_ref, w_ref, b_ref, hres_ref, outh_ref, outp_ref = refs
    elif variant == "relu":
        dis_ref, g_ref, p_ref, w_ref, b_ref, outh_ref, outp_ref = refs
    else:
        dis_ref, g_ref, p_ref, w_ref, b_ref, out_ref = refs
    dis = dis_ref[...]
    z = dis * (g_ref[0] + g_ref[1] + p_ref[...])
    y = jnp.dot(z, w_ref[...], preferred_element_type=jnp.float32) + b_ref[...]
    if variant == "plain":
        out_ref[...] = y
    elif variant == "relu":
        a = jnp.maximum(y, 0.0)
        outh_ref[...] = a
        outp_ref[...] = dis * a
    else:
        a = jnp.maximum(y, 0.0)
        hnew = (hres_ref[...] + a) * 0.5
        outh_ref[...] = hnew
        outp_ref[...] = dis * hnew


def _tc_layer(variant, dis2, garr, p, w, b, hres=None):
    nblk = N // BN
    in_specs = [
        pl.BlockSpec((BN, 1), lambda i: (i, 0)),
        pl.BlockSpec((2, BN, D), lambda i: (0, i, 0)),
        pl.BlockSpec((BN, D), lambda i: (i, 0)),
        pl.BlockSpec((D, D), lambda i: (0, 0)),
        pl.BlockSpec((1, D), lambda i: (0, 0)),
    ]
    args = [dis2, garr, p, w, b.reshape(1, D)]
    if variant == "res":
        in_specs.append(pl.BlockSpec((BN, D), lambda i: (i, 0)))
        args.append(hres)
    if variant == "plain":
        out_shape = jax.ShapeDtypeStruct((N, D), jnp.float32)
        out_specs = pl.BlockSpec((BN, D), lambda i: (i, 0))
    else:
        out_shape = (jax.ShapeDtypeStruct((N, D), jnp.float32),
                     jax.ShapeDtypeStruct((N, D), jnp.float32))
        out_specs = (pl.BlockSpec((BN, D), lambda i: (i, 0)),
                     pl.BlockSpec((BN, D), lambda i: (i, 0)))
    return pl.pallas_call(
        functools.partial(_tc_body, variant),
        grid=(nblk,),
        in_specs=in_specs,
        out_specs=out_specs,
        out_shape=out_shape,
    )(*args)


def kernel(x, edge_index, weights, biases):
    row = edge_index[0]
    col = edge_index[1]

    degp = _deg_kernel(col)
    deg = degp[:N] + degp[N:] + 1.0
    dis2 = lax.rsqrt(deg)[:, None]

    def agg(p):
        g = _agg_kernel(p, row, col)
        return g.reshape(NC, N, D)

    p = dis2 * x
    h, p = _tc_layer("relu", dis2, agg(p), p, weights[0], biases[0])
    wi = 1
    for _ in range(3):
        t, pt = _tc_layer("relu", dis2, agg(p), p, weights[wi], biases[wi])
        h, p = _tc_layer("res", dis2, agg(pt), pt, weights[wi + 1],
                         biases[wi + 1], hres=h)
        wi += 2
    out = _tc_layer("plain", dis2, agg(p), p, weights[wi], biases[wi])
    return (out, h)


# speedup vs baseline: 20.5483x; 2.3554x over previous
import functools

import jax
import jax.numpy as jnp
from jax import lax
from jax.experimental import pallas as pl
from jax.experimental.pallas import tpu as pltpu
from jax.experimental.pallas import tpu_sc as plsc

N = 10000
E = 320000
D = 128
NC = 2
NS = 16
NW = NC * NS
EW = E // NW
CH = 80
ITERS = EW // CH
ROWS_PER_SUB = N // NS
STG = 25
NSTG = ROWS_PER_SUB // STG
DEG_CH = 624
DEG_REM = N - NS * DEG_CH

_mesh = plsc.VectorSubcoreMesh(core_axis_name="c", subcore_axis_name="s")


def _fill1d(ref, n, value):
    vec = jnp.full((16,), value, jnp.float32)

    def body(i, carry):
        ref[pl.ds(i * 16, 16)] = vec
        return carry

    lax.fori_loop(0, n // 16, body, 0)


def _fill2d(ref, rows, value):
    vec = jnp.full((16,), value, jnp.float32)

    def rbody(r, carry):
        for cidx in range(D // 16):
            ref[r, pl.ds(cidx * 16, 16)] = vec
        return carry

    lax.fori_loop(0, rows, rbody, 0)


@functools.partial(
    pl.kernel,
    out_type=jax.ShapeDtypeStruct((NC * N,), jnp.float32),
    mesh=_mesh,
    compiler_params=pltpu.CompilerParams(use_tc_tiling_on_sc=False),
    scratch_types=[
        pltpu.VMEM_SHARED((N,), jnp.float32),
        pltpu.VMEM((CH,), jnp.int32),
        pltpu.VMEM((CH,), jnp.float32),
        pltpu.VMEM((DEG_CH,), jnp.float32),
    ],
)
def _deg_kernel(col_hbm, deg_out, acc, colbuf, onesbuf, stage):
    c = lax.axis_index("c")
    s = lax.axis_index("s")
    wid = c * NS + s

    _fill1d(stage, DEG_CH, 0.0)
    _fill1d(onesbuf, CH, 1.0)
    pltpu.sync_copy(stage, acc.at[pl.ds(s * DEG_CH, DEG_CH)])

    @pl.when(s == NS - 1)
    def _zero_tail():
        pltpu.sync_copy(stage.at[pl.ds(0, DEG_REM)],
                        acc.at[pl.ds(NS * DEG_CH, DEG_REM)])

    plsc.subcore_barrier()

    def body(i, carry):
        base = wid * EW + i * CH
        pltpu.sync_copy(col_hbm.at[pl.ds(base, CH)], colbuf)
        pltpu.sync_copy(onesbuf, acc.at[colbuf], add=True)
        return carry

    lax.fori_loop(0, ITERS, body, 0)
    plsc.subcore_barrier()

    pltpu.sync_copy(acc.at[pl.ds(s * DEG_CH, DEG_CH)], stage)
    pltpu.sync_copy(stage, deg_out.at[pl.ds(c * N + s * DEG_CH, DEG_CH)])

    @pl.when(s == NS - 1)
    def _copy_tail():
        pltpu.sync_copy(acc.at[pl.ds(NS * DEG_CH, DEG_REM)],
                        stage.at[pl.ds(0, DEG_REM)])
        pltpu.sync_copy(stage.at[pl.ds(0, DEG_REM)],
                        deg_out.at[pl.ds(c * N + NS * DEG_CH, DEG_REM)])


@functools.partial(
    pl.kernel,
    out_type=jax.ShapeDtypeStruct((NC * N, D), jnp.float32),
    mesh=_mesh,
    compiler_params=pltpu.CompilerParams(use_tc_tiling_on_sc=False),
    scratch_types=[
        pltpu.VMEM_SHARED((N, D), jnp.float32),
        pltpu.VMEM((ITERS, CH), jnp.int32),
        pltpu.VMEM((ITERS, CH), jnp.int32),
        pltpu.VMEM((CH, D), jnp.float32),
        pltpu.VMEM((CH, D), jnp.float32),
        pltpu.VMEM((STG, D), jnp.float32),
        pltpu.SemaphoreType.DMA,
        pltpu.SemaphoreType.DMA,
        pltpu.SemaphoreType.DMA,
        pltpu.SemaphoreType.DMA,
        pltpu.SemaphoreType.DMA,
    ],
)
def _agg_kernel(p_hbm, row_hbm, col_hbm, g_out,
                acc, rowidx, colidx, buf0, buf1, stage, sg0, sg1, ss0, ss1,
                sidx):
    c = lax.axis_index("c")
    s = lax.axis_index("s")
    wid = c * NS + s
    bufs = (buf0, buf1)
    sgs = (sg0, sg1)
    sss = (ss0, ss1)

    di_r = pltpu.async_copy(row_hbm.at[wid], rowidx, sidx)
    di_c = pltpu.async_copy(col_hbm.at[wid], colidx, sidx)

    _fill2d(stage, STG, 0.0)

    def zbody(k, carry):
        pltpu.sync_copy(stage, acc.at[pl.ds(s * ROWS_PER_SUB + k * STG, STG)])
        return carry

    lax.fori_loop(0, NSTG, zbody, 0)
    di_r.wait()
    di_c.wait()
    plsc.subcore_barrier()

    pltpu.async_copy(p_hbm.at[rowidx.at[0]], buf0, sg0)
    pltpu.async_copy(p_hbm.at[rowidx.at[1]], buf1, sg1)

    def chunk(k, b):
        buf, sg, ss = bufs[b], sgs[b], sss[b]
        pltpu.make_async_copy(p_hbm.at[rowidx.at[k]], buf, sg).wait()
        pltpu.async_copy(buf, acc.at[colidx.at[k]], ss, add=True)
        pltpu.make_async_copy(buf, acc.at[colidx.at[k]], ss).wait()

        @pl.when(k + 2 < ITERS)
        def _next_gather():
            pltpu.async_copy(p_hbm.at[rowidx.at[k + 2]], buf, sg)

    def loop_body(k2, carry):
        chunk(2 * k2, 0)
        chunk(2 * k2 + 1, 1)
        return carry

    lax.fori_loop(0, ITERS // 2, loop_body, 0)
    chunk(ITERS - 1, 0)
    plsc.subcore_barrier()

    base = s * ROWS_PER_SUB
    pltpu.sync_copy(acc.at[pl.ds(base, ROWS_PER_SUB)],
                    g_out.at[pl.ds(c * N + base, ROWS_PER_SUB)])


BN = 1000


def _tc_body(variant, *refs):
    if variant == "res":
        dis_ref, g_ref, p_ref, w_ref, b_ref, hres_ref, outh_ref, outp_ref = refs
    elif variant == "relu":
        dis_ref, g_ref, p_ref, w_ref, b_ref, outh_ref, outp_ref = refs
    else:
        dis_ref, g_ref, p_ref, w_ref, b_ref, out_ref = refs
    dis = dis_ref[...]
    z = dis * (g_ref[0] + g_ref[1] + p_ref[...])
    y = jnp.dot(z, w_ref[...], preferred_element_type=jnp.float32) + b_ref[...]
    if variant == "plain":
        out_ref[...] = y
    elif variant == "relu":
        a = jnp.maximum(y, 0.0)
        outh_ref[...] = a
        outp_ref[...] = dis * a
    else:
        a = jnp.maximum(y, 0.0)
        hnew = (hres_ref[...] + a) * 0.5
        outh_ref[...] = hnew
        outp_ref[...] = dis * hnew


def _tc_layer(variant, dis2, garr, p, w, b, hres=None):
    nblk = N // BN
    in_specs = [
        pl.BlockSpec((BN, 1), lambda i: (i, 0)),
        pl.BlockSpec((2, BN, D), lambda i: (0, i, 0)),
        pl.BlockSpec((BN, D), lambda i: (i, 0)),
        pl.BlockSpec((D, D), lambda i: (0, 0)),
        pl.BlockSpec((1, D), lambda i: (0, 0)),
    ]
    args = [dis2, garr, p, w, b.reshape(1, D)]
    if variant == "res":
        in_specs.append(pl.BlockSpec((BN, D), lambda i: (i, 0)))
        args.append(hres)
    if variant == "plain":
        out_shape = jax.ShapeDtypeStruct((N, D), jnp.float32)
        out_specs = pl.BlockSpec((BN, D), lambda i: (i, 0))
    else:
        out_shape = (jax.ShapeDtypeStruct((N, D), jnp.float32),
                     jax.ShapeDtypeStruct((N, D), jnp.float32))
        out_specs = (pl.BlockSpec((BN, D), lambda i: (i, 0)),
                     pl.BlockSpec((BN, D), lambda i: (i, 0)))
    return pl.pallas_call(
        functools.partial(_tc_body, variant),
        grid=(nblk,),
        in_specs=in_specs,
        out_specs=out_specs,
        out_shape=out_shape,
    )(*args)


def kernel(x, edge_index, weights, biases):
    row = edge_index[0]
    col = edge_index[1]

    degp = _deg_kernel(col)
    deg = degp[:N] + degp[N:] + 1.0
    dis2 = lax.rsqrt(deg)[:, None]

    row3 = row.reshape(NW, ITERS, CH)
    col3 = col.reshape(NW, ITERS, CH)

    def agg(p):
        g = _agg_kernel(p, row3, col3)
        return g.reshape(NC, N, D)

    p = dis2 * x
    h, p = _tc_layer("relu", dis2, agg(p), p, weights[0], biases[0])
    wi = 1
    for _ in range(3):
        t, pt = _tc_layer("relu", dis2, agg(p), p, weights[wi], biases[wi])
        h, p = _tc_layer("res", dis2, agg(pt), pt, weights[wi + 1],
                         biases[wi + 1], hres=h)
        wi += 2
    out = _tc_layer("plain", dis2, agg(p), p, weights[wi], biases[wi])
    return (out, h)
